# baseline (speedup 1.0000x reference)
import functools
import math

import jax
import jax.numpy as jnp
from jax import lax
from jax.experimental import pallas as pl
from jax.experimental.pallas import tpu as pltpu
from jax.experimental.pallas import tpu_sc as plsc

_L = 16
_GB = 128
_NBUF = 2


@functools.lru_cache(maxsize=None)
def _build_gather(T, B, V, D):
    info = plsc.get_sparse_core_info()
    nc, ns = info.num_cores, info.num_subcores
    nw = nc * ns
    n_groups = (T * B) // _GB
    assert n_groups % nw == 0
    g_per_w = n_groups // nw
    assert g_per_w % _NBUF == 0
    tb_per_t = B // _GB
    d_tiles = D // 8
    scale = math.sqrt(D)

    mesh = plsc.VectorSubcoreMesh(core_axis_name="c", subcore_axis_name="s")

    @functools.partial(
        pl.kernel,
        out_type=jax.ShapeDtypeStruct((T, d_tiles, tb_per_t, 8, _GB), jnp.float32),
        mesh=mesh,
        scratch_types=[
            [pltpu.VMEM((_GB,), jnp.int32) for _ in range(_NBUF)],
            [pltpu.VMEM((_GB, D), jnp.float32) for _ in range(_NBUF)],
            pltpu.VMEM((D, _GB), jnp.float32),
            [pltpu.SemaphoreType.DMA for _ in range(_NBUF)],
            pltpu.SemaphoreType.DMA,
        ],
        compiler_params=pltpu.CompilerParams(
            use_tc_tiling_on_sc=False, needs_layout_passes=False
        ),
    )
    def gather_kernel(idx_hbm, table_hbm, out_hbm, idx_v, rows_v, blk_v, gsem, osem):
        wid = lax.axis_index("s") * nc + lax.axis_index("c")
        gbase = wid * g_per_w

        lanes = lax.iota(jnp.int32, _L)

        for b in range(_NBUF):
            pltpu.sync_copy(idx_hbm.at[pl.ds((gbase + b) * _GB, _GB)], idx_v[b])
            pltpu.async_copy(table_hbm.at[idx_v[b]], rows_v[b], gsem[b])

        @pl.loop(0, g_per_w, step=_NBUF)
        def _grp(gi):
            for b in range(_NBUF):
                g = gbase + gi + b
                pltpu.make_async_copy(table_hbm.at[idx_v[b]], rows_v[b], gsem[b]).wait()

                @plsc.parallel_loop(0, _GB // _L)
                def _bb(bb):
                    b0 = bb * _L
                    for d0 in range(0, D, _L):
                        for k in range(_L):
                            dd = d0 + ((lanes + k) & (_L - 1))
                            bv = b0 + lanes
                            v = plsc.load_gather(rows_v[b], [bv, dd])
                            plsc.store_scatter(blk_v, [dd, bv], v * scale)

                t = g // tb_per_t
                tb = g % tb_per_t
                for td in range(d_tiles):
                    pltpu.async_copy(
                        blk_v.at[pl.ds(td * 8, 8), :], out_hbm.at[t, td, tb], osem
                    )
                for td in range(d_tiles):
                    pltpu.make_async_copy(
                        blk_v.at[pl.ds(td * 8, 8), :], out_hbm.at[t, td, tb], osem
                    ).wait()

                nxt = gbase + gi + b + _NBUF

                @pl.when(nxt < gbase + g_per_w)
                def _prefetch():
                    pltpu.sync_copy(idx_hbm.at[pl.ds(nxt * _GB, _GB)], idx_v[b])
                    pltpu.async_copy(table_hbm.at[idx_v[b]], rows_v[b], gsem[b])

    return gather_kernel


def kernel(x, table):
    Bm, T = x.shape
    V, D = table.shape
    xt = jnp.transpose(x).reshape(Bm * T).astype(jnp.int32)
    out5 = _build_gather(T, Bm, V, D)(xt, table)
    tb_per_t = Bm // _GB
    out = out5.transpose(2, 4, 0, 1, 3).reshape(Bm, T, D)
    return out


# speedup vs baseline: 1.3611x; 1.3611x over previous
import functools
import math

import jax
import jax.numpy as jnp
from jax import lax
from jax.experimental import pallas as pl
from jax.experimental.pallas import tpu as pltpu
from jax.experimental.pallas import tpu_sc as plsc

_L = 16
_NBUF = 2


@functools.lru_cache(maxsize=None)
def _build_gather(B, V, D, chunk):
    info = plsc.get_sparse_core_info()
    nc, ns = info.num_cores, info.num_subcores
    nw = nc * ns
    assert B % (nw * chunk) == 0
    b_per_w = B // nw
    n_chunks = b_per_w // chunk
    assert n_chunks % _NBUF == 0
    scale = math.sqrt(D)
    DP = 128

    mesh = plsc.VectorSubcoreMesh(core_axis_name="c", subcore_axis_name="s")

    @functools.partial(
        pl.kernel,
        out_type=jax.ShapeDtypeStruct((B, DP), jnp.float32),
        mesh=mesh,
        scratch_types=[
            [pltpu.VMEM((chunk,), jnp.int32) for _ in range(_NBUF)],
            [pltpu.VMEM((chunk, D), jnp.float32) for _ in range(_NBUF)],
            [pltpu.VMEM((chunk, DP), jnp.float32) for _ in range(_NBUF)],
            [pltpu.SemaphoreType.DMA for _ in range(_NBUF)],
            [pltpu.SemaphoreType.DMA for _ in range(_NBUF)],
        ],
        compiler_params=pltpu.CompilerParams(
            use_tc_tiling_on_sc=False, needs_layout_passes=False
        ),
    )
    def gather_kernel(idx_hbm, table_hbm, out_hbm, idx_v, rows_v, pad_v, gsem, ssem):
        wid = lax.axis_index("s") * nc + lax.axis_index("c")
        base = wid * b_per_w

        def start_gather(b, cb):
            pltpu.sync_copy(idx_hbm.at[pl.ds(cb, chunk)], idx_v[b])
            pltpu.async_copy(table_hbm.at[idx_v[b]], rows_v[b], gsem[b])

        for b in range(_NBUF):
            start_gather(b, base + b * chunk)

        @pl.loop(0, n_chunks, step=_NBUF)
        def _group(g):
            for b in range(_NBUF):
                i = g + b
                pltpu.make_async_copy(table_hbm.at[idx_v[b]], rows_v[b], gsem[b]).wait()

                out_prev = out_hbm.at[pl.ds(base + (i - _NBUF) * chunk, chunk)]

                @pl.when(i >= _NBUF)
                def _wait_prev():
                    pltpu.make_async_copy(pad_v[b], out_prev, ssem[b]).wait()

                @plsc.parallel_loop(0, chunk, unroll=2)
                def _row(r):
                    for j in range(D // _L):
                        sl = pl.ds(j * _L, _L)
                        pad_v[b][r, sl] = rows_v[b][r, sl] * scale

                out_slice = out_hbm.at[pl.ds(base + i * chunk, chunk)]
                pltpu.async_copy(pad_v[b], out_slice, ssem[b])

                nxt = i + _NBUF

                @pl.when(nxt < n_chunks)
                def _prefetch():
                    start_gather(b, base + nxt * chunk)

                @pl.when(nxt >= n_chunks)
                def _drain():
                    pltpu.make_async_copy(pad_v[b], out_slice, ssem[b]).wait()

    return gather_kernel


def kernel(x, table):
    B = x.shape[0] * x.shape[1]
    V, D = table.shape
    xf = x.reshape(B).astype(jnp.int32)
    out_p = _build_gather(B, V, D, 256)(xf, table)
    out = out_p[:, :D].reshape(x.shape + (D,))
    return out


# speedup vs baseline: 1.4797x; 1.0871x over previous
import functools
import math

import jax
import jax.numpy as jnp
from jax import lax
from jax.experimental import pallas as pl
from jax.experimental.pallas import tpu as pltpu
from jax.experimental.pallas import tpu_sc as plsc

_L = 16
_NBUF = 2


@functools.lru_cache(maxsize=None)
def _build_gather(B, V, D, chunk):
    info = plsc.get_sparse_core_info()
    nc, ns = info.num_cores, info.num_subcores
    nw = nc * ns
    assert B % (nw * chunk) == 0
    b_per_w = B // nw
    n_chunks = b_per_w // chunk
    assert n_chunks % _NBUF == 0
    scale = math.sqrt(D)
    DP = 128

    mesh = plsc.VectorSubcoreMesh(core_axis_name="c", subcore_axis_name="s")

    @functools.partial(
        pl.kernel,
        out_type=jax.ShapeDtypeStruct((B, DP), jnp.float32),
        mesh=mesh,
        scratch_types=[
            [pltpu.VMEM((chunk,), jnp.int32) for _ in range(_NBUF)],
            [pltpu.VMEM((chunk, D), jnp.float32) for _ in range(_NBUF)],
            [pltpu.SemaphoreType.DMA for _ in range(_NBUF)],
            [pltpu.SemaphoreType.DMA for _ in range(_NBUF)],
        ],
        compiler_params=pltpu.CompilerParams(
            use_tc_tiling_on_sc=False, needs_layout_passes=False
        ),
    )
    def gather_kernel(idx_hbm, table_hbm, out_hbm, idx_v, rows_v, gsem, ssem):
        wid = lax.axis_index("s") * nc + lax.axis_index("c")
        base = wid * b_per_w

        def start_gather(b, cb):
            pltpu.sync_copy(idx_hbm.at[pl.ds(cb, chunk)], idx_v[b])
            pltpu.async_copy(table_hbm.at[idx_v[b]], rows_v[b], gsem[b])

        def out_window(i):
            return out_hbm.at[pl.ds(base + i * chunk, chunk), pl.ds(0, D)]

        for b in range(_NBUF):
            start_gather(b, base + b * chunk)

        @pl.loop(0, n_chunks, step=_NBUF)
        def _group(g):
            for b in range(_NBUF):
                i = g + b
                pltpu.make_async_copy(table_hbm.at[idx_v[b]], rows_v[b], gsem[b]).wait()

                @plsc.parallel_loop(0, chunk, unroll=2)
                def _row(r):
                    for j in range(D // _L):
                        sl = pl.ds(j * _L, _L)
                        rows_v[b][r, sl] = rows_v[b][r, sl] * scale

                pltpu.async_copy(rows_v[b], out_window(i), ssem[b])

                nxt = i + _NBUF

                @pl.when(nxt < n_chunks)
                def _prefetch():
                    pltpu.sync_copy(
                        idx_hbm.at[pl.ds(base + nxt * chunk, chunk)], idx_v[b]
                    )
                    pltpu.make_async_copy(rows_v[b], out_window(i), ssem[b]).wait()
                    pltpu.async_copy(table_hbm.at[idx_v[b]], rows_v[b], gsem[b])

                @pl.when(nxt >= n_chunks)
                def _drain():
                    pltpu.make_async_copy(rows_v[b], out_window(i), ssem[b]).wait()

    return gather_kernel


def kernel(x, table):
    B = x.shape[0] * x.shape[1]
    V, D = table.shape
    xf = x.reshape(B).astype(jnp.int32)
    out_p = _build_gather(B, V, D, 800)(xf, table)
    out = out_p[:, :D].reshape(x.shape + (D,))
    return out


# speedup vs baseline: 1.4818x; 1.0014x over previous
import functools
import math

import jax
import jax.numpy as jnp
from jax import lax
from jax.experimental import pallas as pl
from jax.experimental.pallas import tpu as pltpu
from jax.experimental.pallas import tpu_sc as plsc

_L = 16
_NBUF = 2


@functools.lru_cache(maxsize=None)
def _build_gather(B, V, D, chunk):
    info = plsc.get_sparse_core_info()
    nc, ns = info.num_cores, info.num_subcores
    nw = nc * ns
    assert B % (nw * chunk) == 0
    b_per_w = B // nw
    n_chunks = b_per_w // chunk
    assert n_chunks % _NBUF == 0
    scale = math.sqrt(D)
    DP = 128

    mesh = plsc.VectorSubcoreMesh(core_axis_name="c", subcore_axis_name="s")

    @functools.partial(
        pl.kernel,
        out_type=jax.ShapeDtypeStruct((B, DP), jnp.float32),
        mesh=mesh,
        scratch_types=[
            pltpu.VMEM((b_per_w,), jnp.int32),
            [pltpu.VMEM((chunk, D), jnp.float32) for _ in range(_NBUF)],
            [pltpu.SemaphoreType.DMA for _ in range(_NBUF)],
            [pltpu.SemaphoreType.DMA for _ in range(_NBUF)],
        ],
        compiler_params=pltpu.CompilerParams(
            use_tc_tiling_on_sc=False, needs_layout_passes=False
        ),
    )
    def gather_kernel(idx_hbm, table_hbm, out_hbm, idx_v, rows_v, gsem, ssem):
        wid = lax.axis_index("s") * nc + lax.axis_index("c")
        base = wid * b_per_w

        pltpu.sync_copy(idx_hbm.at[pl.ds(base, b_per_w)], idx_v)

        def start_gather(b, i):
            pltpu.async_copy(
                table_hbm.at[idx_v.at[pl.ds(i * chunk, chunk)]], rows_v[b], gsem[b]
            )

        def out_window(i):
            return out_hbm.at[pl.ds(base + i * chunk, chunk), pl.ds(0, D)]

        for b in range(_NBUF):
            start_gather(b, b)

        @pl.loop(0, n_chunks, step=_NBUF)
        def _group(g):
            for b in range(_NBUF):
                i = g + b
                pltpu.make_async_copy(
                    table_hbm.at[idx_v.at[pl.ds(i * chunk, chunk)]], rows_v[b], gsem[b]
                ).wait()

                @plsc.parallel_loop(0, chunk, unroll=2)
                def _row(r):
                    for j in range(D // _L):
                        sl = pl.ds(j * _L, _L)
                        rows_v[b][r, sl] = rows_v[b][r, sl] * scale

                pltpu.async_copy(rows_v[b], out_window(i), ssem[b])

                nxt = i + _NBUF

                @pl.when(nxt < n_chunks)
                def _prefetch():
                    pltpu.make_async_copy(rows_v[b], out_window(i), ssem[b]).wait()
                    start_gather(b, nxt)

                @pl.when(nxt >= n_chunks)
                def _drain():
                    pltpu.make_async_copy(rows_v[b], out_window(i), ssem[b]).wait()

    return gather_kernel


def kernel(x, table):
    B = x.shape[0] * x.shape[1]
    V, D = table.shape
    xf = x.reshape(B).astype(jnp.int32)
    out_p = _build_gather(B, V, D, 800)(xf, table)
    out = out_p[:, :D].reshape(x.shape + (D,))
    return out
